# Initial kernel scaffold; baseline (speedup 1.0000x reference)
import jax
import jax.numpy as jnp
from jax.experimental import pallas as pl


def kernel(dense_features, sparse_features, emb_tables, dense_params, over_params):
    raise NotImplementedError("write your pallas kernel here")


# speedup vs baseline: 6.8938x; 6.8938x over previous
import functools

import numpy as np
import jax
import jax.numpy as jnp
from jax import lax
from jax.experimental import pallas as pl
from jax.experimental.pallas import tpu as pltpu
from jax.experimental.pallas import tpu_sc as plsc

B = 4096
F = 26
V = 100000
D = 32
NP1 = F + 1
DENSE_IN = 13

_NC = 2
_NS = 16
_NW = _NC * _NS
_IDX_MINOR = 128

_VFOLD = 8
_CW = 51200
_CROWS = _CW // _VFOLD
_NCH = 2
_VQ = _NCH * _CROWS
_NHALF = 2
_HB = F * _NHALF
_GROWS = B * F * _NHALF
_ROWS_PER_W = _GROWS // (_NW * _IDX_MINOR)


def _tc_repack(emb_tables):
    tt = jnp.transpose(emb_tables, (0, 2, 1)).reshape(_HB, D // 2, V)

    def body(in_ref, out_ref):
        a = in_ref[0]
        b = jnp.concatenate(
            [a[:, q * _CROWS : (q + 1) * _CROWS] for q in range(_VFOLD)], axis=0
        )
        out_ref[0] = jnp.transpose(b)

    return pl.pallas_call(
        body,
        grid=(_HB, _NCH),
        in_specs=[pl.BlockSpec((1, D // 2, _CW), lambda g, c: (g, 0, c))],
        out_specs=pl.BlockSpec((1, _CROWS, _VFOLD * 16), lambda g, c: (g, c, 0)),
        out_shape=jax.ShapeDtypeStruct((_HB, _VQ, _VFOLD * 16), jnp.float32),
    )(tt)


def _sc_gather(idx3, table16):
    mesh = plsc.VectorSubcoreMesh(core_axis_name="c", subcore_axis_name="s")

    @functools.partial(
        pl.kernel,
        out_type=jax.ShapeDtypeStruct((_NW, _ROWS_PER_W, _IDX_MINOR, 16), jnp.float32),
        mesh=mesh,
        compiler_params=pltpu.CompilerParams(use_tc_tiling_on_sc=False),
        scratch_types=[
            pltpu.VMEM((_ROWS_PER_W, _IDX_MINOR), jnp.int32),
            pltpu.VMEM((_ROWS_PER_W, _IDX_MINOR, 16), jnp.float32),
            pltpu.SemaphoreType.DMA,
        ],
    )
    def gather_kernel(idx_hbm, table_hbm, out_hbm, idx_v, rows_v, sem):
        wid = lax.axis_index("s") * _NC + lax.axis_index("c")
        pltpu.sync_copy(idx_hbm.at[wid], idx_v)
        copies = []
        for j in range(_ROWS_PER_W):
            copies.append(
                pltpu.async_copy(table_hbm.at[idx_v.at[j]], rows_v.at[j], sem)
            )
        for c in copies:
            c.wait()
        pltpu.sync_copy(rows_v, out_hbm.at[wid])

    return gather_kernel(idx3, table16)


_BLK = 512
_NSH = NP1 // 2


def _tc_body(dT_ref, eT_ref, w1T, b1, w2T, b2, w3T, b3,
             oxT, wselT, bo1, o2T, bo2, o3T, bo3, o4T, bo4, outT_ref):
    x = jnp.maximum(jnp.dot(w1T[...], dT_ref[...]) + b1[...], 0.0)
    x = jnp.maximum(jnp.dot(w2T[...], x) + b2[...], 0.0)
    x = jnp.maximum(jnp.dot(w3T[...], x) + b3[...], 0.0)

    comb = jnp.concatenate([x, eT_ref[...]], axis=0)
    pieces = []
    for n in range(1, _NSH + 1):
        rolled = jnp.concatenate([comb[n * D :], comb[: n * D]], axis=0)
        p3 = (comb * rolled).reshape(NP1, D, _BLK)
        s = p3[:, :16] + p3[:, 16:]
        s = s[:, :8] + s[:, 8:]
        s = s[:, :4] + s[:, 4:]
        s = s[:, :2] + s[:, 2:]
        pieces.append(s[:, 0] + s[:, 1])
    inter = jnp.concatenate(pieces, axis=0)

    z = jnp.dot(oxT[...], x) + jnp.dot(wselT[...], inter) + bo1[...]
    z = jnp.maximum(z, 0.0)
    z = jnp.maximum(jnp.dot(o2T[...], z) + bo2[...], 0.0)
    z = jnp.maximum(jnp.dot(o3T[...], z) + bo3[...], 0.0)
    outT_ref[...] = jnp.dot(o4T[...], z) + bo4[...]


def _pair_perm():
    perm = np.empty(13 * NP1, np.int64)
    for n in range(1, _NSH + 1):
        for m in range(NP1):
            a, bb = m, (m + n) % NP1
            i, j = max(a, bb), min(a, bb)
            perm[(n - 1) * NP1 + m] = i * (i - 1) // 2 + j
    return perm


_PERM = _pair_perm()


def kernel(dense_features, sparse_features, emb_tables, dense_params, over_params):
    v = sparse_features.astype(jnp.int32)
    c = v // _CW
    u = v - c * _CW
    q = u // _CROWS
    s = c * _CROWS + (u - q * _CROWS)
    f_off = jnp.arange(F, dtype=jnp.int32)[None, :] * (_NHALF * _VQ * _VFOLD)
    r0 = f_off + s * _VFOLD + q
    ridx = jnp.stack([r0, r0 + _VQ * _VFOLD], axis=2)
    idx3 = ridx.reshape(_NW, _ROWS_PER_W * _IDX_MINOR).reshape(
        _NW, _ROWS_PER_W, _IDX_MINOR
    )

    (w1, b1), (w2, b2), (w3, b3) = dense_params
    (wo1, bo1), (o2, bo2), (o3, bo3), (o4, bo4) = over_params
    oxT = wo1[:D].T
    wselT = wo1[D:][_PERM].T
    col = lambda r: r.reshape(-1, 1)

    table16 = _tc_repack(emb_tables).reshape(_HB * _VQ * _VFOLD, 16)
    gathered = _sc_gather(idx3, table16)
    embT = gathered.reshape(B, F * D).T

    full = lambda shape: pl.BlockSpec(shape, lambda i: (0,) * len(shape))
    outT = pl.pallas_call(
        _tc_body,
        grid=(B // _BLK,),
        in_specs=[
            pl.BlockSpec((DENSE_IN, _BLK), lambda i: (0, i)),
            pl.BlockSpec((F * D, _BLK), lambda i: (0, i)),
            full((512, DENSE_IN)), full((512, 1)),
            full((256, 512)), full((256, 1)),
            full((D, 256)), full((D, 1)),
            full((512, D)), full((512, 13 * NP1)), full((512, 1)),
            full((512, 512)), full((512, 1)),
            full((256, 512)), full((256, 1)),
            full((1, 256)), full((1, 1)),
        ],
        out_specs=pl.BlockSpec((1, _BLK), lambda i: (0, i)),
        out_shape=jax.ShapeDtypeStruct((1, B), jnp.float32),
    )(
        dense_features.T, embT,
        w1.T, col(b1), w2.T, col(b2), w3.T, col(b3),
        oxT, wselT, col(bo1), o2.T, col(bo2), o3.T, col(bo3), o4.T, col(bo4),
    )
    return outT.reshape(B, 1)
